# Initial kernel scaffold; baseline (speedup 1.0000x reference)
import jax
import jax.numpy as jnp
from jax.experimental import pallas as pl


def kernel(node_features, edge_features, targets, edge_index, W_dense, b_dense, W_edge, b_edge, W_out, b_out):
    raise NotImplementedError("write your pallas kernel here")


# speedup vs baseline: 1.6737x; 1.6737x over previous
import functools

import jax
import jax.numpy as jnp
from jax import lax
from jax.experimental import pallas as pl
from jax.experimental.pallas import tpu as pltpu
from jax.experimental.pallas import tpu_sc as plsc

N = 10000
E = 320000
D = 128

NUM_SC = 2
NUM_TILES = 16
NW = NUM_SC * NUM_TILES
EPW = E // NW
CHUNK = 80
NCHUNK = EPW // CHUNK
ROWS_PT = 624
ROWS_REM = N - NUM_TILES * ROWS_PT
ZROWS = 208


def _mish(h):
    return h * jnp.tanh(jax.nn.softplus(h))


def _dense_mish_body(a_ref, w_ref, b_ref, o_ref):
    h = lax.dot_general(a_ref[...], w_ref[...], (((1,), (1,)), ((), ())),
                        preferred_element_type=jnp.float32)
    h = h + b_ref[...]
    o_ref[...] = _mish(h)


def _dense_mish(a, w, b, block_rows):
    rows = a.shape[0]
    grid = rows // block_rows
    return pl.pallas_call(
        _dense_mish_body,
        grid=(grid,),
        in_specs=[
            pl.BlockSpec((block_rows, D), lambda i: (i, 0)),
            pl.BlockSpec((D, D), lambda i: (0, 0)),
            pl.BlockSpec((1, D), lambda i: (0, 0)),
        ],
        out_specs=pl.BlockSpec((block_rows, D), lambda i: (i, 0)),
        out_shape=jax.ShapeDtypeStruct((rows, D), jnp.float32),
    )(a, w, b.reshape(1, D))


def _final_body(x_ref, a0_ref, a1_ref, w_ref, b_ref, o_ref):
    s = x_ref[...] + a0_ref[...] + a1_ref[...]
    h = lax.dot_general(s, w_ref[...], (((1,), (1,)), ((), ())),
                        preferred_element_type=jnp.float32)
    h = h + b_ref[...]
    o_ref[...] = _mish(h)


def _final_dense(x, a0, a1, w, b, block_rows):
    grid = N // block_rows
    return pl.pallas_call(
        _final_body,
        grid=(grid,),
        in_specs=[
            pl.BlockSpec((block_rows, D), lambda i: (i, 0)),
            pl.BlockSpec((block_rows, D), lambda i: (i, 0)),
            pl.BlockSpec((block_rows, D), lambda i: (i, 0)),
            pl.BlockSpec((D, D), lambda i: (0, 0)),
            pl.BlockSpec((1, D), lambda i: (0, 0)),
        ],
        out_specs=pl.BlockSpec((block_rows, D), lambda i: (i, 0)),
        out_shape=jax.ShapeDtypeStruct((N, D), jnp.float32),
    )(x, a0, a1, w, b.reshape(1, D))


def _sc_agg_body(x_hbm, y_hbm, src_hbm, dst_hbm, out_hbm,
                 src_v, dst_v, xrow_v, yrow_v, zero_v, agg_sh, sem):
    c = lax.axis_index("c")
    s = lax.axis_index("s")
    wid = c * NUM_TILES + s

    def zbody(j, _):
        r = j // 8
        col = (j % 8) * 16
        zero_v[r, pl.ds(col, 16)] = jnp.zeros((16,), jnp.float32)
        return 0
    lax.fori_loop(0, ZROWS * 8, zbody, 0)
    rbase = pl.multiple_of(s * ROWS_PT, 16)
    for j in range(ROWS_PT // ZROWS):
        pltpu.sync_copy(zero_v, agg_sh.at[pl.ds(rbase + j * ZROWS, ZROWS)])

    @pl.when(s == NUM_TILES - 1)
    def _():
        pltpu.sync_copy(zero_v.at[pl.ds(0, ROWS_REM)],
                        agg_sh.at[pl.ds(N - ROWS_REM, ROWS_REM)])
    plsc.subcore_barrier()

    ebase = wid * EPW

    def chunk_body(i, _):
        start = pl.multiple_of(ebase + i * CHUNK, 16)
        pltpu.sync_copy(src_hbm.at[pl.ds(start, CHUNK)], src_v)
        pltpu.sync_copy(dst_hbm.at[pl.ds(start, CHUNK)], dst_v)
        pltpu.async_copy(x_hbm.at[src_v], xrow_v, sem).wait()
        pltpu.sync_copy(y_hbm.at[pl.ds(start, CHUNK)], yrow_v)

        def cbody(j, _):
            r = j // 8
            col = (j % 8) * 16
            v = xrow_v[r, pl.ds(col, 16)] + yrow_v[r, pl.ds(col, 16)]
            xrow_v[r, pl.ds(col, 16)] = jnp.maximum(v, 0.0)
            return 0
        lax.fori_loop(0, CHUNK * 8, cbody, 0)

        pltpu.sync_copy(xrow_v, agg_sh.at[dst_v], add=True)
        return 0

    lax.fori_loop(0, NCHUNK, chunk_body, 0)
    plsc.subcore_barrier()

    pltpu.sync_copy(agg_sh.at[pl.ds(rbase, ROWS_PT)],
                    out_hbm.at[c, pl.ds(rbase, ROWS_PT)])

    @pl.when(s == NUM_TILES - 1)
    def _():
        pltpu.sync_copy(agg_sh.at[pl.ds(N - ROWS_REM, ROWS_REM)],
                        out_hbm.at[c, pl.ds(N - ROWS_REM, ROWS_REM)])


_sc_agg = functools.partial(
    pl.kernel,
    out_type=jax.ShapeDtypeStruct((NUM_SC, N, D), jnp.float32),
    mesh=plsc.VectorSubcoreMesh(core_axis_name="c", subcore_axis_name="s"),
    scratch_types=[
        pltpu.VMEM((CHUNK,), jnp.int32),
        pltpu.VMEM((CHUNK,), jnp.int32),
        pltpu.VMEM((CHUNK, D), jnp.float32),
        pltpu.VMEM((CHUNK, D), jnp.float32),
        pltpu.VMEM((ZROWS, D), jnp.float32),
        pltpu.VMEM_SHARED((N, D), jnp.float32),
        pltpu.SemaphoreType.DMA,
    ],
)(_sc_agg_body)


def kernel(node_features, edge_features, targets, edge_index,
           W_dense, b_dense, W_edge, b_edge, W_out, b_out):
    x = _dense_mish(node_features, W_dense, b_dense, 1000)
    y = _dense_mish(edge_features, W_edge, b_edge, 2000)
    src = edge_index[0]
    dst = edge_index[1]
    agg = _sc_agg(x, y, src, dst)
    return _final_dense(x, agg[0], agg[1], W_out, b_out, 1000)


# speedup vs baseline: 3.8036x; 2.2725x over previous
import functools

import jax
import jax.numpy as jnp
from jax import lax
from jax.experimental import pallas as pl
from jax.experimental.pallas import tpu as pltpu
from jax.experimental.pallas import tpu_sc as plsc

N = 10000
E = 320000
D = 128

NUM_SC = 2
NUM_TILES = 16
NW = NUM_SC * NUM_TILES
EPW = E // NW
CHUNK = 64
NCHUNK = EPW // CHUNK
TAIL = EPW - NCHUNK * CHUNK
ROWS_PT = 624
ROWS_REM = N - NUM_TILES * ROWS_PT
ZROWS = 48


def _mish(h):
    return h * jnp.tanh(jax.nn.softplus(h))


def _dense_mish_body(a_ref, w_ref, b_ref, o_ref):
    h = lax.dot_general(a_ref[...], w_ref[...], (((1,), (1,)), ((), ())),
                        preferred_element_type=jnp.float32)
    h = h + b_ref[...]
    o_ref[...] = _mish(h)


def _dense_mish(a, w, b, block_rows):
    rows = a.shape[0]
    grid = rows // block_rows
    return pl.pallas_call(
        _dense_mish_body,
        grid=(grid,),
        in_specs=[
            pl.BlockSpec((block_rows, D), lambda i: (i, 0)),
            pl.BlockSpec((D, D), lambda i: (0, 0)),
            pl.BlockSpec((1, D), lambda i: (0, 0)),
        ],
        out_specs=pl.BlockSpec((block_rows, D), lambda i: (i, 0)),
        out_shape=jax.ShapeDtypeStruct((rows, D), jnp.float32),
    )(a, w, b.reshape(1, D))


def _final_body(x_ref, a0_ref, a1_ref, w_ref, b_ref, o_ref):
    s = x_ref[...] + a0_ref[...] + a1_ref[...]
    h = lax.dot_general(s, w_ref[...], (((1,), (1,)), ((), ())),
                        preferred_element_type=jnp.float32)
    h = h + b_ref[...]
    o_ref[...] = _mish(h)


def _final_dense(x, a0, a1, w, b, block_rows):
    grid = N // block_rows
    return pl.pallas_call(
        _final_body,
        grid=(grid,),
        in_specs=[
            pl.BlockSpec((block_rows, D), lambda i: (i, 0)),
            pl.BlockSpec((block_rows, D), lambda i: (i, 0)),
            pl.BlockSpec((block_rows, D), lambda i: (i, 0)),
            pl.BlockSpec((D, D), lambda i: (0, 0)),
            pl.BlockSpec((1, D), lambda i: (0, 0)),
        ],
        out_specs=pl.BlockSpec((block_rows, D), lambda i: (i, 0)),
        out_shape=jax.ShapeDtypeStruct((N, D), jnp.float32),
    )(x, a0, a1, w, b.reshape(1, D))


def _sc_agg_body(x_hbm, y_hbm, src_hbm, dst_hbm, out_hbm,
                 src0, src1, dst0, dst1, xr0, xr1, yr0, yr1,
                 srcT, dstT, xrT, yrT,
                 semA0, semA1, semB0, semB1, agg_sh):
    c = lax.axis_index("c")
    s = lax.axis_index("s")
    wid = c * NUM_TILES + s
    src_b = (src0, src1)
    dst_b = (dst0, dst1)
    xr_b = (xr0, xr1)
    yr_b = (yr0, yr1)
    semA = (semA0, semA1)
    semB = (semB0, semB1)

    def zbody(r, _):
        for cc in range(8):
            xr0[r, pl.ds(cc * 16, 16)] = jnp.zeros((16,), jnp.float32)
        return 0
    lax.fori_loop(0, ZROWS, zbody, 0)
    rbase = pl.multiple_of(s * ROWS_PT, 16)
    for j in range(ROWS_PT // ZROWS):
        pltpu.sync_copy(xr0.at[pl.ds(0, ZROWS)],
                        agg_sh.at[pl.ds(rbase + j * ZROWS, ZROWS)])

    @pl.when(s == NUM_TILES - 1)
    def _():
        pltpu.sync_copy(xr0.at[pl.ds(0, ROWS_REM)],
                        agg_sh.at[pl.ds(N - ROWS_REM, ROWS_REM)])
    plsc.subcore_barrier()

    ebase = wid * EPW

    def start_idx(k, p):
        st = pl.multiple_of(ebase + k * CHUNK, 16)
        pltpu.async_copy(src_hbm.at[pl.ds(st, CHUNK)], src_b[p], semA[p])
        pltpu.async_copy(dst_hbm.at[pl.ds(st, CHUNK)], dst_b[p], semA[p])

    def wait_idx(p):
        pltpu.make_async_copy(src_hbm.at[pl.ds(0, CHUNK)], src_b[p],
                              semA[p]).wait()
        pltpu.make_async_copy(dst_hbm.at[pl.ds(0, CHUNK)], dst_b[p],
                              semA[p]).wait()

    def start_data(k, p):
        st = pl.multiple_of(ebase + k * CHUNK, 16)
        pltpu.async_copy(x_hbm.at[src_b[p]], xr_b[p], semB[p])
        pltpu.async_copy(y_hbm.at[pl.ds(st, CHUNK)], yr_b[p], semB[p])

    def wait_data(p):
        pltpu.make_async_copy(y_hbm.at[pl.ds(0, CHUNK)], xr_b[p],
                              semB[p]).wait()
        pltpu.make_async_copy(y_hbm.at[pl.ds(0, CHUNK)], yr_b[p],
                              semB[p]).wait()

    start_idx(0, 0)
    start_idx(1, 1)
    wait_idx(0)
    start_data(0, 0)

    def pair_body(i2, _):
        for p in (0, 1):
            k = i2 * 2 + p

            @pl.when(k + 1 < NCHUNK)
            def _():
                wait_idx(p ^ 1)
                start_data(k + 1, p ^ 1)

            wait_data(p)

            def cbody(r, _):
                for cc in range(8):
                    v = (xr_b[p][r, pl.ds(cc * 16, 16)] +
                         yr_b[p][r, pl.ds(cc * 16, 16)])
                    yr_b[p][r, pl.ds(cc * 16, 16)] = jnp.maximum(v, 0.0)
                return 0
            lax.fori_loop(0, CHUNK, cbody, 0)

            pltpu.sync_copy(yr_b[p], agg_sh.at[dst_b[p]], add=True)

            @pl.when(k + 2 < NCHUNK)
            def _():
                start_idx(k + 2, p)
        return 0

    lax.fori_loop(0, NCHUNK // 2, pair_body, 0)

    tst = pl.multiple_of(ebase + NCHUNK * CHUNK, 16)
    pltpu.sync_copy(src_hbm.at[pl.ds(tst, TAIL)], srcT)
    pltpu.sync_copy(dst_hbm.at[pl.ds(tst, TAIL)], dstT)
    pltpu.async_copy(x_hbm.at[srcT], xrT, semB0).wait()
    pltpu.sync_copy(y_hbm.at[pl.ds(tst, TAIL)], yrT)

    def tbody(r, _):
        for cc in range(8):
            v = xrT[r, pl.ds(cc * 16, 16)] + yrT[r, pl.ds(cc * 16, 16)]
            yrT[r, pl.ds(cc * 16, 16)] = jnp.maximum(v, 0.0)
        return 0
    lax.fori_loop(0, TAIL, tbody, 0)
    pltpu.sync_copy(yrT, agg_sh.at[dstT], add=True)

    plsc.subcore_barrier()

    pltpu.sync_copy(agg_sh.at[pl.ds(rbase, ROWS_PT)],
                    out_hbm.at[c, pl.ds(rbase, ROWS_PT)])

    @pl.when(s == NUM_TILES - 1)
    def _():
        pltpu.sync_copy(agg_sh.at[pl.ds(N - ROWS_REM, ROWS_REM)],
                        out_hbm.at[c, pl.ds(N - ROWS_REM, ROWS_REM)])


_sc_agg = functools.partial(
    pl.kernel,
    out_type=jax.ShapeDtypeStruct((NUM_SC, N, D), jnp.float32),
    mesh=plsc.VectorSubcoreMesh(core_axis_name="c", subcore_axis_name="s"),
    scratch_types=[
        pltpu.VMEM((CHUNK,), jnp.int32),
        pltpu.VMEM((CHUNK,), jnp.int32),
        pltpu.VMEM((CHUNK,), jnp.int32),
        pltpu.VMEM((CHUNK,), jnp.int32),
        pltpu.VMEM((CHUNK, D), jnp.float32),
        pltpu.VMEM((CHUNK, D), jnp.float32),
        pltpu.VMEM((CHUNK, D), jnp.float32),
        pltpu.VMEM((CHUNK, D), jnp.float32),
        pltpu.VMEM((TAIL,), jnp.int32),
        pltpu.VMEM((TAIL,), jnp.int32),
        pltpu.VMEM((TAIL, D), jnp.float32),
        pltpu.VMEM((TAIL, D), jnp.float32),
        pltpu.SemaphoreType.DMA,
        pltpu.SemaphoreType.DMA,
        pltpu.SemaphoreType.DMA,
        pltpu.SemaphoreType.DMA,
        pltpu.VMEM_SHARED((N, D), jnp.float32),
    ],
)(_sc_agg_body)


def kernel(node_features, edge_features, targets, edge_index,
           W_dense, b_dense, W_edge, b_edge, W_out, b_out):
    x = _dense_mish(node_features, W_dense, b_dense, 1000)
    y = _dense_mish(edge_features, W_edge, b_edge, 2000)
    src = edge_index[0]
    dst = edge_index[1]
    agg = _sc_agg(x, y, src, dst)
    return _final_dense(x, agg[0], agg[1], W_out, b_out, 1000)


# speedup vs baseline: 4.5181x; 1.1878x over previous
import functools

import jax
import jax.numpy as jnp
from jax import lax
from jax.experimental import pallas as pl
from jax.experimental.pallas import tpu as pltpu
from jax.experimental.pallas import tpu_sc as plsc

N = 10000
E = 320000
D = 128

NUM_SC = 2
NUM_TILES = 16
NW = NUM_SC * NUM_TILES
EPW = E // NW
CHUNK = 64
NCHUNK = EPW // CHUNK
TAIL = EPW - NCHUNK * CHUNK
ROWS_PT = 624
ROWS_REM = N - NUM_TILES * ROWS_PT
ZROWS = 48


def _mish(h):
    return h * jnp.tanh(jax.nn.softplus(h))


def _dense_mish_body(a_ref, w_ref, b_ref, o_ref):
    h = lax.dot_general(a_ref[...], w_ref[...], (((1,), (1,)), ((), ())),
                        preferred_element_type=jnp.float32)
    h = h + b_ref[...]
    o_ref[...] = _mish(h)


def _dense_mish(a, w, b, block_rows):
    rows = a.shape[0]
    grid = rows // block_rows
    return pl.pallas_call(
        _dense_mish_body,
        grid=(grid,),
        in_specs=[
            pl.BlockSpec((block_rows, D), lambda i: (i, 0)),
            pl.BlockSpec((D, D), lambda i: (0, 0)),
            pl.BlockSpec((1, D), lambda i: (0, 0)),
        ],
        out_specs=pl.BlockSpec((block_rows, D), lambda i: (i, 0)),
        out_shape=jax.ShapeDtypeStruct((rows, D), jnp.float32),
    )(a, w, b.reshape(1, D))


def _final_body(x_ref, a0_ref, a1_ref, w_ref, b_ref, o_ref):
    s = x_ref[...] + a0_ref[...] + a1_ref[...]
    h = lax.dot_general(s, w_ref[...], (((1,), (1,)), ((), ())),
                        preferred_element_type=jnp.float32)
    h = h + b_ref[...]
    o_ref[...] = _mish(h)


def _final_dense(x, a0, a1, w, b, block_rows):
    grid = N // block_rows
    return pl.pallas_call(
        _final_body,
        grid=(grid,),
        in_specs=[
            pl.BlockSpec((block_rows, D), lambda i: (i, 0)),
            pl.BlockSpec((block_rows, D), lambda i: (i, 0)),
            pl.BlockSpec((block_rows, D), lambda i: (i, 0)),
            pl.BlockSpec((D, D), lambda i: (0, 0)),
            pl.BlockSpec((1, D), lambda i: (0, 0)),
        ],
        out_specs=pl.BlockSpec((block_rows, D), lambda i: (i, 0)),
        out_shape=jax.ShapeDtypeStruct((N, D), jnp.float32),
    )(x, a0, a1, w, b.reshape(1, D))


def _sc_agg_body(x_hbm, y_hbm, src_hbm, dst_hbm, out_hbm,
                 src0, src1, dst0, dst1, xr0, xr1, yr0, yr1,
                 srcT, dstT, xrT, yrT,
                 semA0, semA1, semB0, semB1, semC0, semC1, agg_sh):
    c = lax.axis_index("c")
    s = lax.axis_index("s")
    wid = c * NUM_TILES + s
    src_b = (src0, src1)
    dst_b = (dst0, dst1)
    xr_b = (xr0, xr1)
    yr_b = (yr0, yr1)
    semA = (semA0, semA1)
    semB = (semB0, semB1)
    semC = (semC0, semC1)

    def zbody(r, _):
        for cc in range(8):
            xr0[r, pl.ds(cc * 16, 16)] = jnp.zeros((16,), jnp.float32)
        return 0
    lax.fori_loop(0, ZROWS, zbody, 0)
    rbase = pl.multiple_of(s * ROWS_PT, 16)
    for j in range(ROWS_PT // ZROWS):
        pltpu.sync_copy(xr0.at[pl.ds(0, ZROWS)],
                        agg_sh.at[pl.ds(rbase + j * ZROWS, ZROWS)])

    @pl.when(s == NUM_TILES - 1)
    def _():
        pltpu.sync_copy(xr0.at[pl.ds(0, ROWS_REM)],
                        agg_sh.at[pl.ds(N - ROWS_REM, ROWS_REM)])
    plsc.subcore_barrier()

    ebase = wid * EPW

    def start_idx(k, p):
        st = pl.multiple_of(ebase + k * CHUNK, 16)
        pltpu.async_copy(src_hbm.at[pl.ds(st, CHUNK)], src_b[p], semA[p])
        pltpu.async_copy(dst_hbm.at[pl.ds(st, CHUNK)], dst_b[p], semA[p])

    def wait_idx(p):
        pltpu.make_async_copy(src_hbm.at[pl.ds(0, CHUNK)], src_b[p],
                              semA[p]).wait()
        pltpu.make_async_copy(dst_hbm.at[pl.ds(0, CHUNK)], dst_b[p],
                              semA[p]).wait()

    def start_data(k, p):
        st = pl.multiple_of(ebase + k * CHUNK, 16)
        pltpu.async_copy(x_hbm.at[src_b[p]], xr_b[p], semB[p])
        pltpu.async_copy(y_hbm.at[pl.ds(st, CHUNK)], yr_b[p], semB[p])

    def wait_data(p):
        pltpu.make_async_copy(y_hbm.at[pl.ds(0, CHUNK)], xr_b[p],
                              semB[p]).wait()
        pltpu.make_async_copy(y_hbm.at[pl.ds(0, CHUNK)], yr_b[p],
                              semB[p]).wait()

    def wait_scatter(p):
        pltpu.make_async_copy(y_hbm.at[pl.ds(0, CHUNK)], yr_b[p],
                              semC[p]).wait()

    start_idx(0, 0)
    start_idx(1, 1)
    wait_idx(0)
    start_data(0, 0)

    def pair_body(i2, _):
        for p in (0, 1):
            k = i2 * 2 + p

            @pl.when(k + 1 < NCHUNK)
            def _():
                wait_idx(p ^ 1)

                @pl.when(k >= 1)
                def _():
                    wait_scatter(p ^ 1)
                start_data(k + 1, p ^ 1)

            wait_data(p)

            def cbody(r, _):
                for cc in range(8):
                    v = (xr_b[p][r, pl.ds(cc * 16, 16)] +
                         yr_b[p][r, pl.ds(cc * 16, 16)])
                    yr_b[p][r, pl.ds(cc * 16, 16)] = jnp.maximum(v, 0.0)
                return 0
            lax.fori_loop(0, CHUNK, cbody, 0)

            pltpu.async_copy(yr_b[p], agg_sh.at[dst_b[p]], semC[p], add=True)

            @pl.when(k + 2 < NCHUNK)
            def _():
                start_idx(k + 2, p)
        return 0

    lax.fori_loop(0, NCHUNK // 2, pair_body, 0)
    wait_scatter(0)
    wait_scatter(1)

    tst = pl.multiple_of(ebase + NCHUNK * CHUNK, 16)
    pltpu.sync_copy(src_hbm.at[pl.ds(tst, TAIL)], srcT)
    pltpu.sync_copy(dst_hbm.at[pl.ds(tst, TAIL)], dstT)
    pltpu.async_copy(x_hbm.at[srcT], xrT, semB0).wait()
    pltpu.sync_copy(y_hbm.at[pl.ds(tst, TAIL)], yrT)

    def tbody(r, _):
        for cc in range(8):
            v = xrT[r, pl.ds(cc * 16, 16)] + yrT[r, pl.ds(cc * 16, 16)]
            yrT[r, pl.ds(cc * 16, 16)] = jnp.maximum(v, 0.0)
        return 0
    lax.fori_loop(0, TAIL, tbody, 0)
    pltpu.sync_copy(yrT, agg_sh.at[dstT], add=True)

    plsc.subcore_barrier()

    pltpu.sync_copy(agg_sh.at[pl.ds(rbase, ROWS_PT)],
                    out_hbm.at[c, pl.ds(rbase, ROWS_PT)])

    @pl.when(s == NUM_TILES - 1)
    def _():
        pltpu.sync_copy(agg_sh.at[pl.ds(N - ROWS_REM, ROWS_REM)],
                        out_hbm.at[c, pl.ds(N - ROWS_REM, ROWS_REM)])


_sc_agg = functools.partial(
    pl.kernel,
    out_type=jax.ShapeDtypeStruct((NUM_SC, N, D), jnp.float32),
    mesh=plsc.VectorSubcoreMesh(core_axis_name="c", subcore_axis_name="s"),
    scratch_types=[
        pltpu.VMEM((CHUNK,), jnp.int32),
        pltpu.VMEM((CHUNK,), jnp.int32),
        pltpu.VMEM((CHUNK,), jnp.int32),
        pltpu.VMEM((CHUNK,), jnp.int32),
        pltpu.VMEM((CHUNK, D), jnp.float32),
        pltpu.VMEM((CHUNK, D), jnp.float32),
        pltpu.VMEM((CHUNK, D), jnp.float32),
        pltpu.VMEM((CHUNK, D), jnp.float32),
        pltpu.VMEM((TAIL,), jnp.int32),
        pltpu.VMEM((TAIL,), jnp.int32),
        pltpu.VMEM((TAIL, D), jnp.float32),
        pltpu.VMEM((TAIL, D), jnp.float32),
        pltpu.SemaphoreType.DMA,
        pltpu.SemaphoreType.DMA,
        pltpu.SemaphoreType.DMA,
        pltpu.SemaphoreType.DMA,
        pltpu.SemaphoreType.DMA,
        pltpu.SemaphoreType.DMA,
        pltpu.VMEM_SHARED((N, D), jnp.float32),
    ],
)(_sc_agg_body)


def kernel(node_features, edge_features, targets, edge_index,
           W_dense, b_dense, W_edge, b_edge, W_out, b_out):
    x = _dense_mish(node_features, W_dense, b_dense, 1000)
    y = _dense_mish(edge_features, W_edge, b_edge, 4000)
    src = edge_index[0]
    dst = edge_index[1]
    agg = _sc_agg(x, y, src, dst)
    return _final_dense(x, agg[0], agg[1], W_out, b_out, 1000)
